# baseline (speedup 1.0000x reference)
import jax
import jax.numpy as jnp
from jax import lax
from jax.experimental import pallas as pl
from jax.experimental.pallas import tpu as pltpu

FIGSIZE = 416.0
IOU_THRESH = 0.1
B, Q, C = 16, 5000, 92

_CONTRACT_MINOR = (((1,), (1,)), ((), ()))
_BIG = 1e30


def _body(logits_ref, boxes_ref, gt_ref, acc_ref):
    b = pl.program_id(0)
    x = logits_ref[0]
    col = lax.broadcasted_iota(jnp.int32, (Q, C), 1)

    x1c = x[:, 1:2]
    viol = ((x > x1c) & (col >= 2) & (col < C - 1)) | ((x >= x1c) & (col == 0))
    e = jnp.exp(x)
    h = jnp.where(viol, _BIG, e)

    ones_row = jnp.ones((1, C), jnp.float32)
    s_row = lax.dot_general(ones_row, h, _CONTRACT_MINOR,
                            preferred_element_type=jnp.float32)
    w1 = jnp.eye(1, C, 1, dtype=jnp.float32)
    x1r = lax.dot_general(w1, x, _CONTRACT_MINOR,
                          preferred_element_type=jnp.float32,
                          precision=lax.Precision.HIGHEST)

    bxq = boxes_ref[0]
    w4 = jnp.eye(4, dtype=jnp.float32)
    btq = lax.dot_general(w4, bxq, _CONTRACT_MINOR,
                          preferred_element_type=jnp.float32,
                          precision=lax.Precision.HIGHEST)
    cx = btq[0:1]
    cy = btq[1:2]
    hw = btq[2:3] * 0.5
    hh = btq[3:4] * 0.5

    bx1 = (cx - hw) * FIGSIZE
    by1 = (cy - hh) * FIGSIZE
    bx2 = (cx + hw) * FIGSIZE
    by2 = (cy + hh) * FIGSIZE

    gx1 = gt_ref[b, 0]
    gy1 = gt_ref[b, 1]
    gx2 = gt_ref[b, 2]
    gy2 = gt_ref[b, 3]

    ix1 = jnp.maximum(bx1, gx1)
    iy1 = jnp.maximum(by1, gy1)
    ix2 = jnp.minimum(bx2, gx2)
    iy2 = jnp.minimum(by2, gy2)
    inter = jnp.maximum(ix2 - ix1, 0.0) * jnp.maximum(iy2 - iy1, 0.0)
    area_a = (bx2 - bx1) * (by2 - by1)
    area_b = (gx2 - gx1) * (gy2 - gy1)
    iou = inter / (area_a + area_b - inter)

    prob = jnp.exp(x1r) / s_row
    prob_c = jnp.clip(prob, 1e-6, 1.0 - 1e-6)
    sp = -jnp.log(1.0 - prob_c)

    maskb = (s_row < _BIG) & (iou >= IOU_THRESH)
    s_det = jnp.sum(jnp.where(maskb, sp * iou, 0.0))
    s_cnt = jnp.sum(jnp.where(maskb, 1.0, 0.0))
    s_prob = jnp.sum(jnp.where(maskb, prob, 0.0))

    lane = lax.broadcasted_iota(jnp.int32, (1, 8, 128), 2)
    row = lax.broadcasted_iota(jnp.int32, (1, 8, 128), 1)
    sel = row == 0
    acc_ref[...] = (jnp.where(sel & (lane == 0), s_det, 0.0)
                    + jnp.where(sel & (lane == 1), s_cnt, 0.0)
                    + jnp.where(sel & (lane == 2), s_prob, 0.0))


@jax.jit
def kernel(pred_logits, pred_boxes, gt):
    acc = pl.pallas_call(
        _body,
        grid=(B,),
        in_specs=[
            pl.BlockSpec((1, Q, C), lambda b: (b, 0, 0)),
            pl.BlockSpec((1, Q, 4), lambda b: (b, 0, 0)),
            pl.BlockSpec(memory_space=pltpu.SMEM),
        ],
        out_specs=pl.BlockSpec((1, 8, 128), lambda b: (b, 0, 0)),
        out_shape=jax.ShapeDtypeStruct((B, 8, 128), jnp.float32),
        compiler_params=pltpu.CompilerParams(
            dimension_semantics=("arbitrary",),
        ),
    )(pred_logits, pred_boxes, gt)

    det_per = acc[:, 0, 0]
    cnt = acc[:, 0, 1]
    psum = acc[:, 0, 2]
    has = cnt > 0
    det_loss = jnp.mean(jnp.where(has, det_per, 0.0))
    max_probs = jnp.where(has, psum / jnp.maximum(cnt, 1.0), 0.0)
    return det_loss, max_probs


# speedup vs baseline: 1.4575x; 1.4575x over previous
import jax
import jax.numpy as jnp
from jax import lax
from jax.experimental import pallas as pl
from jax.experimental.pallas import tpu as pltpu

FIGSIZE = 416.0
IOU_THRESH = 0.1
B, Q, C = 16, 5000, 92

_CONTRACT_MINOR = (((1,), (1,)), ((), ()))
_BIG = 1e30


def _body(logits_ref, boxes_ref, gt_ref, acc_ref):
    b = pl.program_id(0)
    x = logits_ref[0]
    col = lax.broadcasted_iota(jnp.int32, (Q, C), 1)

    x1c = x[:, 1:2]
    viol = ((x > x1c) & (col >= 2) & (col < C - 1)) | ((x >= x1c) & (col == 0))
    e = jnp.exp(x)
    h = jnp.where(viol, _BIG, e)

    ones_row = jnp.ones((1, C), jnp.float32)
    s_row = lax.dot_general(ones_row, h, _CONTRACT_MINOR,
                            preferred_element_type=jnp.float32)
    w1 = jnp.eye(1, C, 1, dtype=jnp.float32)
    x1r = lax.dot_general(w1, x, _CONTRACT_MINOR,
                          preferred_element_type=jnp.float32)

    bxq = boxes_ref[0]
    w4 = jnp.eye(4, dtype=jnp.float32)
    btq = lax.dot_general(w4, bxq, _CONTRACT_MINOR,
                          preferred_element_type=jnp.float32)
    cx = btq[0:1]
    cy = btq[1:2]
    hw = btq[2:3] * 0.5
    hh = btq[3:4] * 0.5

    bx1 = (cx - hw) * FIGSIZE
    by1 = (cy - hh) * FIGSIZE
    bx2 = (cx + hw) * FIGSIZE
    by2 = (cy + hh) * FIGSIZE

    gx1 = gt_ref[b, 0]
    gy1 = gt_ref[b, 1]
    gx2 = gt_ref[b, 2]
    gy2 = gt_ref[b, 3]

    ix1 = jnp.maximum(bx1, gx1)
    iy1 = jnp.maximum(by1, gy1)
    ix2 = jnp.minimum(bx2, gx2)
    iy2 = jnp.minimum(by2, gy2)
    inter = jnp.maximum(ix2 - ix1, 0.0) * jnp.maximum(iy2 - iy1, 0.0)
    area_a = (bx2 - bx1) * (by2 - by1)
    area_b = (gx2 - gx1) * (gy2 - gy1)
    iou = inter / (area_a + area_b - inter)

    prob = jnp.exp(x1r) / s_row
    prob_c = jnp.clip(prob, 1e-6, 1.0 - 1e-6)
    sp = -jnp.log(1.0 - prob_c)

    maskb = (s_row < _BIG) & (iou >= IOU_THRESH)
    s_det = jnp.sum(jnp.where(maskb, sp * iou, 0.0))
    s_cnt = jnp.sum(jnp.where(maskb, 1.0, 0.0))
    s_prob = jnp.sum(jnp.where(maskb, prob, 0.0))

    lane = lax.broadcasted_iota(jnp.int32, (1, 8, 128), 2)
    row = lax.broadcasted_iota(jnp.int32, (1, 8, 128), 1)
    sel = row == 0
    acc_ref[...] = (jnp.where(sel & (lane == 0), s_det, 0.0)
                    + jnp.where(sel & (lane == 1), s_cnt, 0.0)
                    + jnp.where(sel & (lane == 2), s_prob, 0.0))


@jax.jit
def kernel(pred_logits, pred_boxes, gt):
    acc = pl.pallas_call(
        _body,
        grid=(B,),
        in_specs=[
            pl.BlockSpec((1, Q, C), lambda b: (b, 0, 0)),
            pl.BlockSpec((1, Q, 4), lambda b: (b, 0, 0)),
            pl.BlockSpec(memory_space=pltpu.SMEM),
        ],
        out_specs=pl.BlockSpec((1, 8, 128), lambda b: (b, 0, 0)),
        out_shape=jax.ShapeDtypeStruct((B, 8, 128), jnp.float32),
        compiler_params=pltpu.CompilerParams(
            dimension_semantics=("arbitrary",),
        ),
    )(pred_logits, pred_boxes, gt)

    det_per = acc[:, 0, 0]
    cnt = acc[:, 0, 1]
    psum = acc[:, 0, 2]
    has = cnt > 0
    det_loss = jnp.mean(jnp.where(has, det_per, 0.0))
    max_probs = jnp.where(has, psum / jnp.maximum(cnt, 1.0), 0.0)
    return det_loss, max_probs


# speedup vs baseline: 1.6448x; 1.1285x over previous
import jax
import jax.numpy as jnp
from jax import lax
from jax.experimental import pallas as pl
from jax.experimental.pallas import tpu as pltpu

FIGSIZE = 416.0
IOU_THRESH = 0.1
B, Q, C = 16, 5000, 92

_CONTRACT_MINOR = (((1,), (1,)), ((), ()))


def _body(logits_ref, boxes_ref, gt_ref, acc_ref):
    b = pl.program_id(0)

    x = logits_ref[0]
    col = lax.broadcasted_iota(jnp.int32, (Q, C), 1)

    x1c = x[:, 1:2]
    e = jnp.exp(x)
    g = jnp.where((x > x1c) & (col >= 2) & (col < C - 1), 1.0, 0.0)

    ones_row = jnp.ones((1, C), jnp.float32)
    s_row = lax.dot_general(ones_row, e, _CONTRACT_MINOR,
                            preferred_element_type=jnp.float32)
    n_row = lax.dot_general(ones_row, g, _CONTRACT_MINOR,
                            preferred_element_type=jnp.float32)
    w2 = jnp.eye(2, C, dtype=jnp.float32)
    x01 = lax.dot_general(w2, x, _CONTRACT_MINOR,
                          preferred_element_type=jnp.float32)
    x0r = x01[0:1]
    x1r = x01[1:2]

    bx = boxes_ref[0]
    w4 = jnp.eye(4, dtype=jnp.float32)
    bt = lax.dot_general(w4, bx, _CONTRACT_MINOR,
                         preferred_element_type=jnp.float32)
    cx = bt[0:1]
    cy = bt[1:2]
    hw = bt[2:3] * 0.5
    hh = bt[3:4] * 0.5

    bx1 = (cx - hw) * FIGSIZE
    by1 = (cy - hh) * FIGSIZE
    bx2 = (cx + hw) * FIGSIZE
    by2 = (cy + hh) * FIGSIZE

    gx1 = gt_ref[b, 0]
    gy1 = gt_ref[b, 1]
    gx2 = gt_ref[b, 2]
    gy2 = gt_ref[b, 3]

    ix1 = jnp.maximum(bx1, gx1)
    iy1 = jnp.maximum(by1, gy1)
    ix2 = jnp.minimum(bx2, gx2)
    iy2 = jnp.minimum(by2, gy2)
    inter = jnp.maximum(ix2 - ix1, 0.0) * jnp.maximum(iy2 - iy1, 0.0)
    area_a = (bx2 - bx1) * (by2 - by1)
    area_b = (gx2 - gx1) * (gy2 - gy1)
    iou = inter / (area_a + area_b - inter)

    prob = jnp.exp(x1r) / s_row
    prob_c = jnp.clip(prob, 1e-6, 1.0 - 1e-6)
    sp = -jnp.log(1.0 - prob_c)

    maskb = (x1r > x0r) & (n_row == 0.0) & (iou >= IOU_THRESH)
    s_det = jnp.sum(jnp.where(maskb, sp * iou, 0.0))
    s_cnt = jnp.sum(jnp.where(maskb, 1.0, 0.0))
    s_prob = jnp.sum(jnp.where(maskb, prob, 0.0))

    lane = lax.broadcasted_iota(jnp.int32, (1, 8, 128), 2)
    row = lax.broadcasted_iota(jnp.int32, (1, 8, 128), 1)
    sel = row == 0
    acc_ref[...] = (jnp.where(sel & (lane == 0), s_det, 0.0)
                    + jnp.where(sel & (lane == 1), s_cnt, 0.0)
                    + jnp.where(sel & (lane == 2), s_prob, 0.0))


@jax.jit
def kernel(pred_logits, pred_boxes, gt):
    acc = pl.pallas_call(
        _body,
        grid=(B,),
        in_specs=[
            pl.BlockSpec((1, Q, C), lambda b: (b, 0, 0)),
            pl.BlockSpec((1, Q, 4), lambda b: (b, 0, 0)),
            pl.BlockSpec(memory_space=pltpu.SMEM),
        ],
        out_specs=pl.BlockSpec((1, 8, 128), lambda b: (b, 0, 0)),
        out_shape=jax.ShapeDtypeStruct((B, 8, 128), jnp.float32),
        compiler_params=pltpu.CompilerParams(
            dimension_semantics=("arbitrary",),
        ),
    )(pred_logits, pred_boxes, gt)

    det_per = acc[:, 0, 0]
    cnt = acc[:, 0, 1]
    psum = acc[:, 0, 2]
    has = cnt > 0
    det_loss = jnp.mean(jnp.where(has, det_per, 0.0))
    max_probs = jnp.where(has, psum / jnp.maximum(cnt, 1.0), 0.0)
    return det_loss, max_probs
